# baseline (speedup 1.0000x reference)
import numpy as np
import jax
import jax.numpy as jnp
from jax.experimental import pallas as pl
from jax.experimental.pallas import tpu as pltpu

_STRIDE = 8.0
_H = 128
_W = 128
_NA = 3
_NCLS = 80
_PER_PLANE = _H * _W
_R = 2048


def _decode_body(bbox_ref, conf_ref, cls_ref, anchors_ref,
                 xywha_ref, idx_ref, confs_ref):
    g = pl.program_id(0)
    chunks_per_plane = _PER_PLANE // _R
    a = (g // chunks_per_plane) % _NA
    base = (g % chunks_per_plane) * _R

    t = bbox_ref[0]
    sig = jax.nn.sigmoid(t)
    col = jax.lax.broadcasted_iota(jnp.int32, (_R, 5), 1)
    pos = base + jax.lax.broadcasted_iota(jnp.int32, (_R, 5), 0)
    mesh_x = (pos % _W).astype(jnp.float32)
    mesh_y = (pos // _W).astype(jnp.float32)
    mesh = jnp.where(col == 0, mesh_x, mesh_y)
    xy = (sig + mesh) * _STRIDE
    aw = jnp.where(a == 0, anchors_ref[0, 0],
                   jnp.where(a == 1, anchors_ref[1, 0], anchors_ref[2, 0]))
    ah = jnp.where(a == 0, anchors_ref[0, 1],
                   jnp.where(a == 1, anchors_ref[1, 1], anchors_ref[2, 1]))
    wh = jnp.exp(t) * jnp.where(col == 2, aw, ah)
    ang = (sig * (2.0 * np.pi) - np.pi) * (180.0 / np.pi)
    xywha_ref[0] = jnp.where(col < 2, xy, jnp.where(col < 4, wh, ang))

    c = cls_ref[0]
    m = jnp.max(c, axis=1, keepdims=True)
    lane = jax.lax.broadcasted_iota(jnp.int32, c.shape, 1)
    first_max = jnp.min(jnp.where(c == m, lane, jnp.int32(_NCLS)),
                        axis=1, keepdims=True)
    idx_ref[0] = first_max
    confs_ref[0] = jax.nn.sigmoid(conf_ref[0]) * jax.nn.sigmoid(m)


def kernel(bbox, conf, cls, anchors, img_size):
    nB, nA, nH, nW, _ = bbox.shape
    n_cls = cls.shape[-1]
    n_cells = nB * nA * nH * nW
    G = n_cells // _R

    bbox3 = bbox.reshape(G, _R, 5)
    conf3 = conf.reshape(G, _R, 1)
    cls3 = cls.reshape(G, _R, n_cls)

    xywha, idx, confs = pl.pallas_call(
        _decode_body,
        grid=(G,),
        in_specs=[
            pl.BlockSpec((1, _R, 5), lambda g: (g, 0, 0)),
            pl.BlockSpec((1, _R, 1), lambda g: (g, 0, 0)),
            pl.BlockSpec((1, _R, n_cls), lambda g: (g, 0, 0)),
            pl.BlockSpec((_NA, 2), lambda g: (0, 0)),
        ],
        out_specs=[
            pl.BlockSpec((1, _R, 5), lambda g: (g, 0, 0)),
            pl.BlockSpec((1, _R, 1), lambda g: (g, 0, 0)),
            pl.BlockSpec((1, _R, 1), lambda g: (g, 0, 0)),
        ],
        out_shape=[
            jax.ShapeDtypeStruct((G, _R, 5), jnp.float32),
            jax.ShapeDtypeStruct((G, _R, 1), jnp.int32),
            jax.ShapeDtypeStruct((G, _R, 1), jnp.float32),
        ],
        compiler_params=pltpu.CompilerParams(
            dimension_semantics=("arbitrary",),
        ),
    )(bbox3, conf3, cls3, anchors)

    flat = nA * nH * nW
    return (xywha.reshape(nB, flat, 5),
            idx.reshape(nB, flat),
            confs.reshape(nB, flat))


# speedup vs baseline: 1.4451x; 1.4451x over previous
import numpy as np
import jax
import jax.numpy as jnp
from jax.experimental import pallas as pl
from jax.experimental.pallas import tpu as pltpu

_STRIDE = 8.0
_H = 128
_W = 128
_NA = 3
_NCLS = 80
_YCHUNK = 16
_R = _YCHUNK * _W
_CPP = _H // _YCHUNK

def _decode_body(bbox_ref, conf_ref, cls_ref, anchors_ref,
                 xywha_ref, idx_ref, confs_ref):
    g = pl.program_id(0)
    a = (g // _CPP) % _NA
    q = g % _CPP

    li = jax.lax.broadcasted_iota(jnp.int32, (1, _W * 5), 1)
    x = li // 5
    col = li - 5 * x
    x8 = x.astype(jnp.float32) * _STRIDE
    col0 = col == 0
    col01 = col < 2
    col2 = col == 2
    col4 = col == 4
    col1f = (col == 1).astype(jnp.float32)

    t = bbox_ref[0]
    e = jnp.exp(t)
    sig = e * (1.0 / (1.0 + e))
    rowy8 = jax.lax.broadcasted_iota(
        jnp.int32, (_YCHUNK, 1), 0).astype(jnp.float32) * _STRIDE
    y8 = rowy8 + (q * (_YCHUNK * _STRIDE)).astype(jnp.float32)
    base = jnp.where(col0, x8, y8 * col1f)
    xy = sig * _STRIDE + base
    aw = jnp.where(a == 0, anchors_ref[0, 0],
                   jnp.where(a == 1, anchors_ref[1, 0], anchors_ref[2, 0]))
    ah = jnp.where(a == 0, anchors_ref[0, 1],
                   jnp.where(a == 1, anchors_ref[1, 1], anchors_ref[2, 1]))
    wh = e * jnp.where(col2, aw, ah)
    ang = sig * 360.0 - 180.0
    xywha_ref[0] = jnp.where(col01, xy, jnp.where(col4, ang, wh))

    c = cls_ref[0]
    m = jnp.max(c, axis=1, keepdims=True)
    lane = jax.lax.broadcasted_iota(jnp.int32, c.shape, 1)
    first_max = jnp.min(jnp.where(c == m, lane, jnp.int32(_NCLS)),
                        axis=1, keepdims=True)
    idx_ref[0] = first_max.reshape(_YCHUNK, _W)
    mm = m.reshape(_YCHUNK, _W)
    cf = conf_ref[0]
    em = jnp.exp(mm)
    ec = jnp.exp(cf)
    confs_ref[0] = (em * ec) * (1.0 / ((1.0 + em) * (1.0 + ec)))


def kernel(bbox, conf, cls, anchors, img_size):
    nB, nA, nH, nW, _ = bbox.shape
    n_cls = cls.shape[-1]
    planes = nB * nA
    G = planes * _CPP

    bbox3 = bbox.reshape(planes, nH, nW * 5)
    conf3 = conf.reshape(planes, nH, nW)
    cls3 = cls.reshape(planes, nH * nW, n_cls)

    xywha, idx, confs = pl.pallas_call(
        _decode_body,
        grid=(G,),
        in_specs=[
            pl.BlockSpec((1, _YCHUNK, nW * 5), lambda g: (g // _CPP, g % _CPP, 0)),
            pl.BlockSpec((1, _YCHUNK, nW), lambda g: (g // _CPP, g % _CPP, 0)),
            pl.BlockSpec((1, _R, n_cls), lambda g: (g // _CPP, g % _CPP, 0)),
            pl.BlockSpec((_NA, 2), lambda g: (0, 0)),
        ],
        out_specs=[
            pl.BlockSpec((1, _YCHUNK, nW * 5), lambda g: (g // _CPP, g % _CPP, 0)),
            pl.BlockSpec((1, _YCHUNK, nW), lambda g: (g // _CPP, g % _CPP, 0)),
            pl.BlockSpec((1, _YCHUNK, nW), lambda g: (g // _CPP, g % _CPP, 0)),
        ],
        out_shape=[
            jax.ShapeDtypeStruct((planes, nH, nW * 5), jnp.float32),
            jax.ShapeDtypeStruct((planes, nH, nW), jnp.int32),
            jax.ShapeDtypeStruct((planes, nH, nW), jnp.float32),
        ],
        compiler_params=pltpu.CompilerParams(
            dimension_semantics=("arbitrary",),
        ),
    )(bbox3, conf3, cls3, anchors)

    flat = nA * nH * nW
    return (xywha.reshape(nB, flat, 5),
            idx.reshape(nB, flat),
            confs.reshape(nB, flat))


# speedup vs baseline: 1.9175x; 1.3269x over previous
import numpy as np
import jax
import jax.numpy as jnp
from jax.experimental import pallas as pl
from jax.experimental.pallas import tpu as pltpu

_STRIDE = 8.0
_H = 128
_W = 128
_NA = 3
_NCLS = 80
_YCHUNK = 16
_R = _YCHUNK * _W
_CPP = _H // _YCHUNK

def _decode_body(bbox_ref, conf_ref, cls_ref, anchors_ref,
                 xywha_ref, idx_ref, confs_ref):
    g = pl.program_id(0)
    a = (g // _CPP) % _NA
    q = g % _CPP

    li = jax.lax.broadcasted_iota(jnp.int32, (1, _W * 5), 1)
    x = li // 5
    col = li - 5 * x
    x8 = x.astype(jnp.float32) * _STRIDE
    col0 = col == 0
    col01 = col < 2
    col2 = col == 2
    col4 = col == 4
    col1f = (col == 1).astype(jnp.float32)

    t = bbox_ref[0]
    e = jnp.exp(t)
    sig = e * (1.0 / (1.0 + e))
    rowy8 = jax.lax.broadcasted_iota(
        jnp.int32, (_YCHUNK, 1), 0).astype(jnp.float32) * _STRIDE
    y8 = rowy8 + (q * (_YCHUNK * _STRIDE)).astype(jnp.float32)
    base = jnp.where(col0, x8, y8 * col1f)
    xy = sig * _STRIDE + base
    aw = jnp.where(a == 0, anchors_ref[0, 0],
                   jnp.where(a == 1, anchors_ref[1, 0], anchors_ref[2, 0]))
    ah = jnp.where(a == 0, anchors_ref[0, 1],
                   jnp.where(a == 1, anchors_ref[1, 1], anchors_ref[2, 1]))
    wh = e * jnp.where(col2, aw, ah)
    ang = sig * 360.0 - 180.0
    xywha_ref[0] = jnp.where(col01, xy, jnp.where(col4, ang, wh))

    c = cls_ref[0]
    ct = c.T
    m = jnp.max(ct, axis=0, keepdims=True)
    sub = jax.lax.broadcasted_iota(jnp.int32, ct.shape, 0)
    first_max = jnp.min(jnp.where(ct == m, sub, jnp.int32(_NCLS)),
                        axis=0)
    idx_ref[0] = first_max.reshape(_YCHUNK, _W)
    m = m.reshape(_R, 1)
    mm = m.reshape(_YCHUNK, _W)
    cf = conf_ref[0]
    em = jnp.exp(mm)
    ec = jnp.exp(cf)
    confs_ref[0] = (em * ec) * (1.0 / ((1.0 + em) * (1.0 + ec)))


def kernel(bbox, conf, cls, anchors, img_size):
    nB, nA, nH, nW, _ = bbox.shape
    n_cls = cls.shape[-1]
    planes = nB * nA
    G = planes * _CPP

    bbox3 = bbox.reshape(planes, nH, nW * 5)
    conf3 = conf.reshape(planes, nH, nW)
    cls3 = cls.reshape(planes, nH * nW, n_cls)

    xywha, idx, confs = pl.pallas_call(
        _decode_body,
        grid=(G,),
        in_specs=[
            pl.BlockSpec((1, _YCHUNK, nW * 5), lambda g: (g // _CPP, g % _CPP, 0)),
            pl.BlockSpec((1, _YCHUNK, nW), lambda g: (g // _CPP, g % _CPP, 0)),
            pl.BlockSpec((1, _R, n_cls), lambda g: (g // _CPP, g % _CPP, 0)),
            pl.BlockSpec((_NA, 2), lambda g: (0, 0)),
        ],
        out_specs=[
            pl.BlockSpec((1, _YCHUNK, nW * 5), lambda g: (g // _CPP, g % _CPP, 0)),
            pl.BlockSpec((1, _YCHUNK, nW), lambda g: (g // _CPP, g % _CPP, 0)),
            pl.BlockSpec((1, _YCHUNK, nW), lambda g: (g // _CPP, g % _CPP, 0)),
        ],
        out_shape=[
            jax.ShapeDtypeStruct((planes, nH, nW * 5), jnp.float32),
            jax.ShapeDtypeStruct((planes, nH, nW), jnp.int32),
            jax.ShapeDtypeStruct((planes, nH, nW), jnp.float32),
        ],
        compiler_params=pltpu.CompilerParams(
            dimension_semantics=("arbitrary",),
        ),
    )(bbox3, conf3, cls3, anchors)

    flat = nA * nH * nW
    return (xywha.reshape(nB, flat, 5),
            idx.reshape(nB, flat),
            confs.reshape(nB, flat))
